# baseline (speedup 1.0000x reference)
import jax
import jax.numpy as jnp
from jax.experimental import pallas as pl
from jax.experimental.pallas import tpu as pltpu

DIM = 128
NUM_EPISODES = 100000
Q = 32
K = 3
CHUNK = 25000
NEG = float("-inf")
BIGI = 2**31 - 1


def _retrieve_kernel(states_ref, context_ref, W1_ref, b1_ref, W2_ref, b2_ref,
                     Wc_ref, bc_ref, query_ref, bank_ref,
                     vals_ref, idx_ref, ectx_ref,
                     rv_ref, ri_ref):
    i = pl.program_id(0)

    query = query_ref[...]
    qt = query.T
    qnsq = jnp.sum(qt * qt, axis=0, keepdims=True).T
    qn = jnp.where(qnsq == 0.0, 0.0, qnsq * jax.lax.rsqrt(qnsq))
    lane8 = jax.lax.broadcasted_iota(jnp.int32, (Q, 8), 1)

    @pl.when(i == 0)
    def _prologue():
        ep = jnp.mean(states_ref[...], axis=0, keepdims=True)
        h = jnp.maximum(
            jnp.dot(ep, W1_ref[...], preferred_element_type=jnp.float32)
            + b1_ref[...], 0.0)
        enc = (jnp.dot(h, W2_ref[...], preferred_element_type=jnp.float32)
               + b2_ref[...])
        ectx_ref[...] = (jnp.dot(context_ref[...], Wc_ref[...],
                                 preferred_element_type=jnp.float32)
                         + bc_ref[...])
        ensq = jnp.sum(enc * enc)
        en = jnp.where(ensq == 0.0, 0.0, ensq * jax.lax.rsqrt(ensq))
        extra_num = jnp.dot(query, enc.T,
                            preferred_element_type=jnp.float32)
        extra = extra_num / jnp.maximum(qn * en, 1e-8)
        rv_ref[...] = jnp.where(lane8 == 0, extra, NEG)
        ri_ref[...] = jnp.where(lane8 == 0, NUM_EPISODES, 0)

    chunk = bank_ref[...]
    num = jax.lax.dot_general(query, chunk, (((1,), (1,)), ((), ())),
                              preferred_element_type=jnp.float32)
    eye = jnp.eye(DIM, dtype=jnp.float32)
    bt = jax.lax.dot_general(eye, chunk, (((1,), (1,)), ((), ())),
                             preferred_element_type=jnp.float32)
    nsq = jnp.sum(bt * bt, axis=0, keepdims=True)
    bn = jnp.where(nsq == 0.0, 0.0, nsq * jax.lax.rsqrt(nsq))
    den = jnp.maximum(qn * bn, 1e-8)
    sims = num / den

    n_full = CHUNK // 128
    rem = CHUNK - n_full * 128
    r1 = jnp.full((Q, 128), NEG, jnp.float32)
    r2 = r1
    r3 = r1
    t1 = jnp.zeros((Q, 128), jnp.int32)
    t2 = t1
    t3 = t1
    for t in range(n_full + (1 if rem else 0)):
        lo = t * 128
        if t < n_full:
            ut = sims[:, lo:lo + 128]
        else:
            ut = jnp.concatenate(
                [sims[:, lo:CHUNK],
                 jnp.full((Q, 128 - rem), NEG, jnp.float32)], axis=1)
        tc = jnp.full((Q, 128), t, jnp.int32)
        c1 = ut > r1
        a = jnp.minimum(r1, ut)
        ta = jnp.where(c1, t1, tc)
        r1 = jnp.maximum(r1, ut)
        t1 = jnp.where(c1, tc, t1)
        c2 = a > r2
        b = jnp.minimum(r2, a)
        tb = jnp.where(c2, t2, ta)
        r2 = jnp.maximum(r2, a)
        t2 = jnp.where(c2, ta, t2)
        c3 = b > r3
        r3 = jnp.maximum(r3, b)
        t3 = jnp.where(c3, tb, t3)

    col = jax.lax.broadcasted_iota(jnp.int32, (Q, 128), 1)
    V = jnp.concatenate([r1, r2, r3], axis=1)
    E = jnp.concatenate([t1 * 128 + col, t2 * 128 + col,
                         t3 * 128 + col], axis=1)
    cand_v, cand_i = [], []
    for _ in range(K):
        m = jnp.max(V, axis=1, keepdims=True)
        ce = jnp.min(jnp.where(V == m, E, BIGI), axis=1, keepdims=True)
        cand_v.append(m)
        cand_i.append(ce + i * CHUNK)
        V = jnp.where(E == ce, NEG, V)

    rv = rv_ref[...]
    ri = ri_ref[...]
    for r in range(K):
        rv = jnp.where(lane8 == K + r, cand_v[r], rv)
        ri = jnp.where(lane8 == K + r, cand_i[r], ri)
    new_v = jnp.full((Q, 8), NEG, jnp.float32)
    new_i = jnp.zeros((Q, 8), jnp.int32)
    for r in range(K):
        m = jnp.max(rv, axis=1, keepdims=True)
        gi = jnp.min(jnp.where(rv == m, ri, BIGI), axis=1, keepdims=True)
        new_v = jnp.where(lane8 == r, m, new_v)
        new_i = jnp.where(lane8 == r, gi, new_i)
        rv = jnp.where(ri == gi, NEG, rv)
    rv_ref[...] = new_v
    ri_ref[...] = new_i
    vals_ref[...] = new_v[:, :K]
    idx_ref[...] = new_i[:, :K]


def kernel(states, context, episode_reprs, query, W1, b1, W2, b2, Wc, bc,
           top_k):
    del top_k
    n_steps = NUM_EPISODES // CHUNK
    const2 = lambda i: (0, 0)
    out = pl.pallas_call(
        _retrieve_kernel,
        grid=(n_steps,),
        in_specs=[
            pl.BlockSpec(states.shape, const2),
            pl.BlockSpec((1, DIM), const2),
            pl.BlockSpec(W1.shape, const2),
            pl.BlockSpec((1, 2 * DIM), const2),
            pl.BlockSpec(W2.shape, const2),
            pl.BlockSpec((1, DIM), const2),
            pl.BlockSpec(Wc.shape, const2),
            pl.BlockSpec((1, DIM), const2),
            pl.BlockSpec(query.shape, const2),
            pl.BlockSpec((CHUNK, DIM), lambda i: (i, 0)),
        ],
        out_specs=[
            pl.BlockSpec((Q, K), const2),
            pl.BlockSpec((Q, K), const2),
            pl.BlockSpec((1, DIM), const2),
        ],
        out_shape=[
            jax.ShapeDtypeStruct((Q, K), jnp.float32),
            jax.ShapeDtypeStruct((Q, K), jnp.int32),
            jax.ShapeDtypeStruct((1, DIM), jnp.float32),
        ],
        scratch_shapes=[
            pltpu.VMEM((Q, 8), jnp.float32),
            pltpu.VMEM((Q, 8), jnp.int32),
        ],
        compiler_params=pltpu.CompilerParams(
            dimension_semantics=("arbitrary",)),
    )(states, context.reshape(1, DIM), W1, b1.reshape(1, -1), W2,
      b2.reshape(1, -1), Wc, bc.reshape(1, -1), query, episode_reprs)
    top_vals, top_idx, ectx = out
    return top_vals, top_idx, ectx.reshape(DIM)


# speedup vs baseline: 1.2341x; 1.2341x over previous
import jax
import jax.numpy as jnp
from jax.experimental import pallas as pl
from jax.experimental.pallas import tpu as pltpu

DIM = 128
NUM_EPISODES = 100000
Q = 32
K = 3
CHUNK = 25000
NEG = float("-inf")
BIGI = 2**31 - 1


def _retrieve_kernel(states_ref, context_ref, W1_ref, b1_ref, W2_ref, b2_ref,
                     Wc_ref, bc_ref, query_ref, bank_ref,
                     vals_ref, idx_ref, ectx_ref,
                     rv_ref, ri_ref):
    i = pl.program_id(0)

    query = query_ref[...]
    qt = query.T
    qnsq = jnp.sum(qt * qt, axis=0, keepdims=True).T
    qn = jnp.where(qnsq == 0.0, 0.0, qnsq * jax.lax.rsqrt(qnsq))
    lane8 = jax.lax.broadcasted_iota(jnp.int32, (Q, 8), 1)

    @pl.when(i == 0)
    def _prologue():
        ep = jnp.mean(states_ref[...], axis=0, keepdims=True)
        h = jnp.maximum(
            jnp.dot(ep, W1_ref[...], preferred_element_type=jnp.float32)
            + b1_ref[...], 0.0)
        enc = (jnp.dot(h, W2_ref[...], preferred_element_type=jnp.float32)
               + b2_ref[...])
        ectx_ref[...] = (jnp.dot(context_ref[...], Wc_ref[...],
                                 preferred_element_type=jnp.float32)
                         + bc_ref[...])
        ensq = jnp.sum(enc * enc)
        en = jnp.where(ensq == 0.0, 0.0, ensq * jax.lax.rsqrt(ensq))
        extra_num = jnp.dot(query, enc.T,
                            preferred_element_type=jnp.float32)
        extra = extra_num / jnp.maximum(qn * en, 1e-8)
        rv_ref[...] = jnp.where(lane8 == 0, extra, NEG)
        ri_ref[...] = jnp.where(lane8 == 0, NUM_EPISODES, 0)

    chunk = bank_ref[...]
    num = jax.lax.dot_general(query, chunk, (((1,), (1,)), ((), ())),
                              preferred_element_type=jnp.float32)
    bt = chunk.T
    nsq = jnp.sum(bt * bt, axis=0, keepdims=True)
    bn = jnp.where(nsq == 0.0, 0.0, nsq * jax.lax.rsqrt(nsq))
    den = jnp.maximum(qn * bn, 1e-8)
    sims = num / den

    n_full = CHUNK // 128
    rem = CHUNK - n_full * 128
    r1 = jnp.full((Q, 128), NEG, jnp.float32)
    r2 = r1
    r3 = r1
    t1 = jnp.zeros((Q, 128), jnp.int32)
    t2 = t1
    t3 = t1
    for t in range(n_full + (1 if rem else 0)):
        lo = t * 128
        if t < n_full:
            ut = sims[:, lo:lo + 128]
        else:
            ut = jnp.concatenate(
                [sims[:, lo:CHUNK],
                 jnp.full((Q, 128 - rem), NEG, jnp.float32)], axis=1)
        tc = jnp.full((Q, 128), t, jnp.int32)
        c1 = ut > r1
        a = jnp.minimum(r1, ut)
        ta = jnp.where(c1, t1, tc)
        r1 = jnp.maximum(r1, ut)
        t1 = jnp.where(c1, tc, t1)
        c2 = a > r2
        b = jnp.minimum(r2, a)
        tb = jnp.where(c2, t2, ta)
        r2 = jnp.maximum(r2, a)
        t2 = jnp.where(c2, ta, t2)
        c3 = b > r3
        r3 = jnp.maximum(r3, b)
        t3 = jnp.where(c3, tb, t3)

    col = jax.lax.broadcasted_iota(jnp.int32, (Q, 128), 1)
    V = jnp.concatenate([r1, r2, r3], axis=1)
    E = jnp.concatenate([t1 * 128 + col, t2 * 128 + col,
                         t3 * 128 + col], axis=1)
    cand_v, cand_i = [], []
    for _ in range(K):
        m = jnp.max(V, axis=1, keepdims=True)
        ce = jnp.min(jnp.where(V == m, E, BIGI), axis=1, keepdims=True)
        cand_v.append(m)
        cand_i.append(ce + i * CHUNK)
        V = jnp.where(E == ce, NEG, V)

    rv = rv_ref[...]
    ri = ri_ref[...]
    for r in range(K):
        rv = jnp.where(lane8 == K + r, cand_v[r], rv)
        ri = jnp.where(lane8 == K + r, cand_i[r], ri)
    new_v = jnp.full((Q, 8), NEG, jnp.float32)
    new_i = jnp.zeros((Q, 8), jnp.int32)
    for r in range(K):
        m = jnp.max(rv, axis=1, keepdims=True)
        gi = jnp.min(jnp.where(rv == m, ri, BIGI), axis=1, keepdims=True)
        new_v = jnp.where(lane8 == r, m, new_v)
        new_i = jnp.where(lane8 == r, gi, new_i)
        rv = jnp.where(ri == gi, NEG, rv)
    rv_ref[...] = new_v
    ri_ref[...] = new_i
    vals_ref[...] = new_v[:, :K]
    idx_ref[...] = new_i[:, :K]


def kernel(states, context, episode_reprs, query, W1, b1, W2, b2, Wc, bc,
           top_k):
    del top_k
    n_steps = NUM_EPISODES // CHUNK
    const2 = lambda i: (0, 0)
    out = pl.pallas_call(
        _retrieve_kernel,
        grid=(n_steps,),
        in_specs=[
            pl.BlockSpec(states.shape, const2),
            pl.BlockSpec((1, DIM), const2),
            pl.BlockSpec(W1.shape, const2),
            pl.BlockSpec((1, 2 * DIM), const2),
            pl.BlockSpec(W2.shape, const2),
            pl.BlockSpec((1, DIM), const2),
            pl.BlockSpec(Wc.shape, const2),
            pl.BlockSpec((1, DIM), const2),
            pl.BlockSpec(query.shape, const2),
            pl.BlockSpec((CHUNK, DIM), lambda i: (i, 0)),
        ],
        out_specs=[
            pl.BlockSpec((Q, K), const2),
            pl.BlockSpec((Q, K), const2),
            pl.BlockSpec((1, DIM), const2),
        ],
        out_shape=[
            jax.ShapeDtypeStruct((Q, K), jnp.float32),
            jax.ShapeDtypeStruct((Q, K), jnp.int32),
            jax.ShapeDtypeStruct((1, DIM), jnp.float32),
        ],
        scratch_shapes=[
            pltpu.VMEM((Q, 8), jnp.float32),
            pltpu.VMEM((Q, 8), jnp.int32),
        ],
        compiler_params=pltpu.CompilerParams(
            dimension_semantics=("arbitrary",)),
    )(states, context.reshape(1, DIM), W1, b1.reshape(1, -1), W2,
      b2.reshape(1, -1), Wc, bc.reshape(1, -1), query, episode_reprs)
    top_vals, top_idx, ectx = out
    return top_vals, top_idx, ectx.reshape(DIM)
